# baseline (speedup 1.0000x reference)
import functools

import jax
import jax.numpy as jnp
from jax import lax
from jax.experimental import pallas as pl
from jax.experimental.pallas import tpu as pltpu
from jax.experimental.pallas import tpu_sc as plsc

_B = 16384
_D = 16
_CHUNK = 128


@functools.lru_cache(maxsize=None)
def _build(NC: int, NS: int, V: int):
    NW = NC * NS
    b_per_w = _B // NW
    n_chunks = b_per_w // _CHUNK
    mesh = plsc.VectorSubcoreMesh(core_axis_name="c", subcore_axis_name="s")

    @functools.partial(
        pl.kernel,
        mesh=mesh,
        compiler_params=pltpu.CompilerParams(use_tc_tiling_on_sc=False),
        out_type=(
            jax.ShapeDtypeStruct((_D, _B), jnp.float32),
            jax.ShapeDtypeStruct((_D, _B), jnp.float32),
        ),
        scratch_types=[
            pltpu.VMEM((b_per_w,), jnp.int32),
            pltpu.VMEM((b_per_w,), jnp.int32),
            pltpu.VMEM((_D, b_per_w), jnp.float32),
            pltpu.VMEM((_D, b_per_w), jnp.float32),
            pltpu.SemaphoreType.DMA,
            pltpu.SemaphoreType.DMA,
        ],
    )
    def k(idx_hbm, pidx_hbm, Wt_hbm, Wpt_hbm, out_hbm, pout_hbm,
          idx_v, pidx_v, out_v, pout_v, sem_a, sem_b):
        wid = lax.axis_index("s") * NC + lax.axis_index("c")
        base = wid * b_per_w
        pltpu.sync_copy(idx_hbm.at[pl.ds(base, b_per_w)], idx_v)
        pltpu.sync_copy(pidx_hbm.at[pl.ds(base, b_per_w)], pidx_v)
        copies = []
        for d in range(_D):
            for c in range(n_chunks):
                s = pl.ds(c * _CHUNK, _CHUNK)
                copies.append(pltpu.async_copy(
                    Wt_hbm.at[d].at[idx_v.at[s]], out_v.at[d, s], sem_a))
                copies.append(pltpu.async_copy(
                    Wpt_hbm.at[d].at[pidx_v.at[s]], pout_v.at[d, s], sem_b))
        for cp in copies:
            cp.wait()
        pltpu.sync_copy(out_v, out_hbm.at[:, pl.ds(base, b_per_w)])
        pltpu.sync_copy(pout_v, pout_hbm.at[:, pl.ds(base, b_per_w)])

    return k


def kernel(batch, persona_batch, W, W_persona):
    info = plsc.get_sparse_core_info()
    NC, NS = info.num_cores, info.num_subcores
    V = W.shape[0]
    out_t, pout_t = _build(NC, NS, V)(
        batch.astype(jnp.int32),
        persona_batch.astype(jnp.int32),
        W.T,
        W_persona.T,
    )
    return out_t.T, pout_t.T


# speedup vs baseline: 1.0032x; 1.0032x over previous
import functools

import jax
import jax.numpy as jnp
from jax import lax
from jax.experimental import pallas as pl
from jax.experimental.pallas import tpu as pltpu
from jax.experimental.pallas import tpu_sc as plsc

_B = 16384
_D = 16


@functools.lru_cache(maxsize=None)
def _build(NC: int, NS: int, V: int):
    NW = NC * NS
    half = _B // (NW // _D)
    mesh = plsc.VectorSubcoreMesh(core_axis_name="c", subcore_axis_name="s")

    @functools.partial(
        pl.kernel,
        mesh=mesh,
        compiler_params=pltpu.CompilerParams(use_tc_tiling_on_sc=False),
        out_type=(
            jax.ShapeDtypeStruct((_D, _B), jnp.float32),
            jax.ShapeDtypeStruct((_D, _B), jnp.float32),
        ),
        scratch_types=[
            pltpu.VMEM((half,), jnp.int32),
            pltpu.VMEM((half,), jnp.int32),
            pltpu.VMEM((half,), jnp.float32),
            pltpu.VMEM((half,), jnp.float32),
            pltpu.SemaphoreType.DMA,
            pltpu.SemaphoreType.DMA,
        ],
    )
    def k(Wt_hbm, Wpt_hbm, idx_hbm, pidx_hbm, out_hbm, pout_hbm,
          idx_v, pidx_v, rows_v, prows_v, sem_a, sem_b):
        wid = lax.axis_index("s") * NC + lax.axis_index("c")
        d = lax.shift_right_logical(wid, 1)
        base = jnp.bitwise_and(wid, 1) * half
        pltpu.sync_copy(idx_hbm.at[pl.ds(base, half)], idx_v)
        pltpu.sync_copy(pidx_hbm.at[pl.ds(base, half)], pidx_v)
        ca = pltpu.async_copy(Wt_hbm.at[d].at[idx_v], rows_v, sem_a)
        cb = pltpu.async_copy(Wpt_hbm.at[d].at[pidx_v], prows_v, sem_b)
        ca.wait()
        cb.wait()
        pltpu.sync_copy(rows_v, out_hbm.at[d, pl.ds(base, half)])
        pltpu.sync_copy(prows_v, pout_hbm.at[d, pl.ds(base, half)])

    return k


def kernel(batch, persona_batch, W, W_persona):
    info = plsc.get_sparse_core_info()
    NC, NS = info.num_cores, info.num_subcores
    V = W.shape[0]
    out_t, pout_t = _build(NC, NS, V)(
        W.T,
        W_persona.T,
        batch.astype(jnp.int32),
        persona_batch.astype(jnp.int32),
    )
    return out_t.T, pout_t.T
